# baseline (speedup 1.0000x reference)
import functools

import jax
import jax.numpy as jnp
from jax import lax
from jax.experimental import pallas as pl
from jax.experimental.pallas import tpu as pltpu
from jax.experimental.pallas import tpu_sc as plsc

N_U = 100000
M_I = 100000
SIDE = 100000
SIDE_P = 102400
NP = 2 * SIDE_P
D = 64
E = 600000
NC = 2
NS = 16
CK = 128
BLK = 8
NCH = 304
NBLK = NCH // BLK
EP_SIDE = NCH * CK * NS
W = 20480
PW = 5
RT = W // NS
WD = SIDE_P
RTD = WD // NS

_f32 = jnp.float32
_i32 = jnp.int32


def _mesh():
    return plsc.VectorSubcoreMesh(core_axis_name="c", subcore_axis_name="s")


def _clamp_to_window(dstb, dstl, base, w):
    for v in range(CK // 16):
        sl = pl.ds(v * 16, 16)
        d = dstb[sl]
        loc = d - base
        ok = (loc >= 0) & (loc < w)
        dstl[sl] = jnp.where(ok, loc, w)


def _clamp_to_window2(dstbb, k, dstl, base, w):
    for v in range(CK // 16):
        sl = pl.ds(v * 16, 16)
        d = dstbb[k, sl]
        loc = d - base
        ok = (loc >= 0) & (loc < w)
        dstl[sl] = jnp.where(ok, loc, w)


def _deg_body(dst_hbm, deg_hbm, acc, dstb, dstl, onesb, degb):
    c = lax.axis_index("c")
    s = lax.axis_index("s")

    zf = jnp.zeros((16,), _f32)

    @pl.loop(0, CK)
    def _fill_ones(r):
        onesb[r, pl.ds(0, 16)] = jnp.full((16,), 1.0, _f32)

    @pl.loop(0, 64)
    def _zb(r):
        degb[r, pl.ds(0, 16)] = zf

    @pl.loop(0, RTD // 64)
    def _zero(i):
        pltpu.sync_copy(degb, acc.at[pl.ds(s * RTD + i * 64, 64)])

    plsc.subcore_barrier()

    @pl.loop(0, NCH)
    def _chunks(j):
        pltpu.sync_copy(dst_hbm.at[c, s, j], dstb)
        _clamp_to_window(dstb, dstl, 0, WD)
        pltpu.sync_copy(onesb, acc.at[dstl], add=True)

    plsc.subcore_barrier()

    pltpu.sync_copy(acc.at[pl.ds(s * RTD, RTD)],
                    deg_hbm.at[pl.ds(c * SIDE_P + s * RTD, RTD)])


def _deg_call(dst2):
    kern = pl.kernel(
        _deg_body,
        out_type=jax.ShapeDtypeStruct((NP, 16), _f32),
        mesh=_mesh(),
        compiler_params=pltpu.CompilerParams(use_tc_tiling_on_sc=False),
        scratch_types=[
            pltpu.VMEM_SHARED((WD + 8, 16), _f32),
            pltpu.VMEM((CK,), _i32),
            pltpu.VMEM((CK,), _i32),
            pltpu.VMEM((CK, 16), _f32),
            pltpu.VMEM((64, 16), _f32),
        ],
    )
    return kern(dst2)


def _scale_body(x_ref, d_ref, o_ref):
    o_ref[:, :] = x_ref[:, :] * d_ref[:, :]


def _u0_call(all_emb_p, dsqi2d):
    return pl.pallas_call(
        _scale_body,
        grid=(NP // 1024,),
        in_specs=[
            pl.BlockSpec((1024, D), lambda i: (i, 0)),
            pl.BlockSpec((1024, 1), lambda i: (i, 0)),
        ],
        out_specs=pl.BlockSpec((1024, D), lambda i: (i, 0)),
        out_shape=jax.ShapeDtypeStruct((NP, D), _f32),
    )(all_emb_p, dsqi2d)


def _layer_body(uprev, src, dst, dinv, unext,
                acc,
                colbb0, colbb1, dstbb0, dstbb1, dstl0, dstl1,
                colb0, colb1,
                gb0, gb1, wb, dinvb,
                gs0, gs1, isem):
    c = lax.axis_index("c")
    s = lax.axis_index("s")
    idxb = ((colbb0, dstbb0), (colbb1, dstbb1))
    gbufs = ((dstl0, colb0, gb0, gs0), (dstl1, colb1, gb1, gs1))

    zf = jnp.zeros((16,), _f32)

    def _row_to_1d(src2d, k, dst1d):
        for v in range(CK // 16):
            sl = pl.ds(v * 16, 16)
            dst1d[sl] = src2d[k, sl]

    pltpu.sync_copy(src.at[c, s, pl.ds(0, BLK)], colbb0)
    pltpu.sync_copy(dst.at[c, s, pl.ds(0, BLK)], dstbb0)
    _row_to_1d(colbb0, 0, colb0)
    _row_to_1d(colbb0, 1, colb1)
    pltpu.async_copy(uprev.at[colb0], gb0, gs0)
    pltpu.async_copy(uprev.at[colb1], gb1, gs1)

    for p in range(PW):
        base = p * W

        @pl.loop(0, 64)
        def _zwb(r):
            for cc in range(D // 16):
                wb[r, pl.ds(cc * 16, 16)] = zf

        @pl.loop(0, RT // 64)
        def _zero(i):
            pltpu.sync_copy(wb, acc.at[pl.ds(s * RT + i * 64, 64)])

        plsc.subcore_barrier()

        @pl.loop(0, NBLK // 2)
        def _blocks(h):
            for ib in range(2):
                i = h * 2 + ib
                colbb, dstbb = idxb[ib]
                ncolbb, ndstbb = idxb[1 - ib]
                inx = lax.rem(i + 1, NBLK)
                cpc = pltpu.async_copy(src.at[c, s, pl.ds(inx * BLK, BLK)],
                                       ncolbb, isem)
                cpd = pltpu.async_copy(dst.at[c, s, pl.ds(inx * BLK, BLK)],
                                       ndstbb, isem)
                for k in range(BLK):
                    if k == BLK - 2:
                        cpc.wait()
                        cpd.wait()
                    b = k % 2
                    dstl, colb, gb, gs = gbufs[b]
                    _clamp_to_window2(dstbb, k, dstl, base, W)
                    pltpu.make_async_copy(uprev.at[colb], gb, gs).wait()
                    pltpu.async_copy(gb, acc.at[dstl], gs, add=True).wait()
                    if k < BLK - 2:
                        _row_to_1d(colbb, k + 2, colb)
                    else:
                        _row_to_1d(ncolbb, k - (BLK - 2), colb)
                    pltpu.async_copy(uprev.at[colb], gb, gs)

        plsc.subcore_barrier()

        g0 = c * SIDE_P + base + s * RT
        pltpu.sync_copy(dinv.at[pl.ds(g0, RT)], dinvb)

        @pl.loop(0, RT // 64)
        def _wb(t):
            pltpu.sync_copy(acc.at[pl.ds(s * RT + t * 64, 64)], wb)

            @pl.loop(0, 4)
            def _rows(g):
                dvv = dinvb[pl.ds(t * 64 + g * 16, 16)]
                for k in range(16):
                    r = g * 16 + k
                    dv = dvv[k]
                    for cc in range(D // 16):
                        sl = pl.ds(cc * 16, 16)
                        wb[r, sl] = wb[r, sl] * dv

            pltpu.sync_copy(wb, unext.at[pl.ds(g0 + t * 64, 64)])

    for b in range(2):
        _, colb, gb, gs = gbufs[b]
        pltpu.make_async_copy(uprev.at[colb], gb, gs).wait()


def _layer_call(uprev, src2, dst2, dinv):
    kern = pl.kernel(
        _layer_body,
        out_type=jax.ShapeDtypeStruct((NP, D), _f32),
        mesh=_mesh(),
        compiler_params=pltpu.CompilerParams(use_tc_tiling_on_sc=False),
        scratch_types=[
            pltpu.VMEM_SHARED((W + 8, D), _f32),
            pltpu.VMEM((BLK, CK), _i32),
            pltpu.VMEM((BLK, CK), _i32),
            pltpu.VMEM((BLK, CK), _i32),
            pltpu.VMEM((BLK, CK), _i32),
            pltpu.VMEM((CK,), _i32),
            pltpu.VMEM((CK,), _i32),
            pltpu.VMEM((CK,), _i32),
            pltpu.VMEM((CK,), _i32),
            pltpu.VMEM((CK, D), _f32),
            pltpu.VMEM((CK, D), _f32),
            pltpu.VMEM((64, D), _f32),
            pltpu.VMEM((RT,), _f32),
            pltpu.SemaphoreType.DMA,
            pltpu.SemaphoreType.DMA,
            pltpu.SemaphoreType.DMA,
        ],
    )
    return kern(uprev, src2, dst2, dinv)


def _final_body(u0, u1, u2, u3, dsq16, uq, iq, out,
                uqb, iqb, gu0, gu1, gu2, gu3, gi0, gi1, gi2, gi3,
                du16, di16, ob16):
    c = lax.axis_index("c")
    s = lax.axis_index("s")
    wid = s * NC + c
    bq = 4096 // (NC * NS)
    sb = 64

    @pl.loop(0, bq // sb)
    def _sub(sr):
        base = wid * bq + sr * sb
        pltpu.sync_copy(uq.at[pl.ds(base, sb)], uqb)
        pltpu.sync_copy(iq.at[pl.ds(base, sb)], iqb)
        for tab, gb in ((u0, gu0), (u1, gu1), (u2, gu2), (u3, gu3)):
            pltpu.sync_copy(tab.at[uqb], gb)
        for tab, gb in ((u0, gi0), (u1, gi1), (u2, gi2), (u3, gi3)):
            pltpu.sync_copy(tab.at[iqb], gb)
        pltpu.sync_copy(dsq16.at[uqb], du16)
        pltpu.sync_copy(dsq16.at[iqb], di16)

        @pl.loop(0, sb)
        def _rows(r):
            tot = jnp.zeros((16,), _f32)
            for cc in range(D // 16):
                sl = pl.ds(cc * 16, 16)
                su = gu0[r, sl] + gu1[r, sl] + gu2[r, sl] + gu3[r, sl]
                si = gi0[r, sl] + gi1[r, sl] + gi2[r, sl] + gi3[r, sl]
                tot = tot + su * si
            ob16[r, pl.ds(0, 16)] = (tot * du16[r, pl.ds(0, 16)]
                                     * di16[r, pl.ds(0, 16)] * 0.0625)

        pltpu.sync_copy(ob16, out.at[pl.ds(base, sb)])


def _final_call(u0, u1, u2, u3, dsq16, uq, iq):
    bq = 4096 // (NC * NS)
    kern = pl.kernel(
        _final_body,
        out_type=jax.ShapeDtypeStruct((4096, 16), _f32),
        mesh=_mesh(),
        compiler_params=pltpu.CompilerParams(use_tc_tiling_on_sc=False),
        scratch_types=[
            pltpu.VMEM((64,), _i32),
            pltpu.VMEM((64,), _i32),
            pltpu.VMEM((64, D), _f32),
            pltpu.VMEM((64, D), _f32),
            pltpu.VMEM((64, D), _f32),
            pltpu.VMEM((64, D), _f32),
            pltpu.VMEM((64, D), _f32),
            pltpu.VMEM((64, D), _f32),
            pltpu.VMEM((64, D), _f32),
            pltpu.VMEM((64, D), _f32),
            pltpu.VMEM((64, 16), _f32),
            pltpu.VMEM((64, 16), _f32),
            pltpu.VMEM((64, 16), _f32),
        ],
    )
    return kern(u0, u1, u2, u3, dsq16, uq, iq)


def kernel(users, items, user_emb, item_emb, edge_u, edge_i, graph_vals):
    eu = edge_u.astype(_i32)
    ei = edge_i.astype(_i32)
    us = users.astype(_i32)
    it = items.astype(_i32)
    pad = EP_SIDE - E
    zpad = jnp.zeros((pad,), _i32)
    npad = jnp.full((pad,), -1, _i32)

    src_u = jnp.concatenate([ei + SIDE_P, zpad])
    dst_u = jnp.concatenate([eu, npad])
    src_i = jnp.concatenate([eu, zpad])
    dst_i = jnp.concatenate([ei, npad])
    src2 = jnp.stack([src_u, src_i]).reshape(NC, NS, NCH, CK)
    dst2 = jnp.stack([dst_u, dst_i]).reshape(NC, NS, NCH, CK)

    rpad = jnp.zeros((SIDE_P - SIDE, D), _f32)
    all_emb_p = jnp.concatenate([user_emb, rpad, item_emb, rpad], axis=0)

    deg = _deg_call(dst2)[:, 0]
    dm = jnp.maximum(deg, 1.0)
    dinv = 1.0 / dm
    dsqi2d = lax.rsqrt(dm)[:, None]
    dsq16 = jnp.broadcast_to(jnp.sqrt(dm)[:, None], (NP, 16))

    u0 = _u0_call(all_emb_p, dsqi2d)
    u1 = _layer_call(u0, src2, dst2, dinv)
    u2 = _layer_call(u1, src2, dst2, dinv)
    u3 = _layer_call(u2, src2, dst2, dinv)

    return jnp.sum(_final_call(u0, u1, u2, u3, dsq16, us, it + SIDE_P), axis=1)


# speedup vs baseline: 5.4935x; 5.4935x over previous
import functools

import jax
import jax.numpy as jnp
from jax import lax
from jax.experimental import pallas as pl
from jax.experimental.pallas import tpu as pltpu
from jax.experimental.pallas import tpu_sc as plsc

N_U = 100000
M_I = 100000
SIDE = 100000
SIDE_P = 102400
NP = 2 * SIDE_P
D = 64
E = 600000
NC = 2
NS = 16
CK = 128
BLK = 8
NCH = 304
NBLK = NCH // BLK
EP_SIDE = NCH * CK * NS
W = 20480
PW = 5
RT = W // NS
WD = SIDE_P
RTD = WD // NS

_f32 = jnp.float32
_i32 = jnp.int32


def _mesh():
    return plsc.VectorSubcoreMesh(core_axis_name="c", subcore_axis_name="s")


def _clamp_to_window(dstb, dstl, base, w):
    for v in range(CK // 16):
        sl = pl.ds(v * 16, 16)
        d = dstb[sl]
        loc = d - base
        ok = (loc >= 0) & (loc < w)
        dstl[sl] = jnp.where(ok, loc, w)


def _clamp_to_window2(dstbb, k, dstl, base, w):
    for v in range(CK // 16):
        sl = pl.ds(v * 16, 16)
        d = dstbb[k, sl]
        loc = d - base
        ok = (loc >= 0) & (loc < w)
        dstl[sl] = jnp.where(ok, loc, w)


def _deg_body(dst_hbm, deg_hbm, acc, dstb, dstl, onesb, degb):
    c = lax.axis_index("c")
    s = lax.axis_index("s")

    zf = jnp.zeros((16,), _f32)

    @pl.loop(0, CK)
    def _fill_ones(r):
        onesb[r, pl.ds(0, 16)] = jnp.full((16,), 1.0, _f32)

    @pl.loop(0, 64)
    def _zb(r):
        degb[r, pl.ds(0, 16)] = zf

    @pl.loop(0, RTD // 64)
    def _zero(i):
        pltpu.sync_copy(degb, acc.at[pl.ds(s * RTD + i * 64, 64)])

    plsc.subcore_barrier()

    @pl.loop(0, NCH)
    def _chunks(j):
        pltpu.sync_copy(dst_hbm.at[c, s, j], dstb)
        _clamp_to_window(dstb, dstl, 0, WD)
        pltpu.sync_copy(onesb, acc.at[dstl], add=True)

    plsc.subcore_barrier()

    pltpu.sync_copy(acc.at[pl.ds(s * RTD, RTD)],
                    deg_hbm.at[pl.ds(c * SIDE_P + s * RTD, RTD)])


def _deg_call(dst2):
    kern = pl.kernel(
        _deg_body,
        out_type=jax.ShapeDtypeStruct((NP, 16), _f32),
        mesh=_mesh(),
        compiler_params=pltpu.CompilerParams(use_tc_tiling_on_sc=False),
        scratch_types=[
            pltpu.VMEM_SHARED((WD + 8, 16), _f32),
            pltpu.VMEM((CK,), _i32),
            pltpu.VMEM((CK,), _i32),
            pltpu.VMEM((CK, 16), _f32),
            pltpu.VMEM((64, 16), _f32),
        ],
    )
    return kern(dst2)


def _scale_body(x_ref, d_ref, o_ref):
    o_ref[:, :] = x_ref[:, :] * d_ref[:, :]


def _u0_call(all_emb_p, dsqi2d):
    return pl.pallas_call(
        _scale_body,
        grid=(NP // 1024,),
        in_specs=[
            pl.BlockSpec((1024, D), lambda i: (i, 0)),
            pl.BlockSpec((1024, 1), lambda i: (i, 0)),
        ],
        out_specs=pl.BlockSpec((1024, D), lambda i: (i, 0)),
        out_shape=jax.ShapeDtypeStruct((NP, D), _f32),
    )(all_emb_p, dsqi2d)


def _layer_body(uprev, bsrc, bdst, cnts, dinv, unext,
                acc,
                colb0, colb1, dstb0, dstb1, dstl0, dstl1,
                gb0, gb1, wb, dinvb, cntb,
                gs0, gs1, ss0, ss1):
    c = lax.axis_index("c")
    s = lax.axis_index("s")
    bufs = ((colb0, dstb0, dstl0, gb0, gs0, ss0),
            (colb1, dstb1, dstl1, gb1, gs1, ss1))

    zf = jnp.zeros((16,), _f32)

    pltpu.sync_copy(cnts.at[c, s], cntb)
    cv = cntb[pl.ds(0, 16)]

    for p in range(PW):
        base = p * W
        nch = cv[p]

        @pl.loop(0, 64)
        def _zwb(r):
            for cc in range(D // 16):
                wb[r, pl.ds(cc * 16, 16)] = zf

        @pl.loop(0, RT // 64)
        def _zero(i):
            pltpu.sync_copy(wb, acc.at[pl.ds(s * RT + i * 64, 64)])

        plsc.subcore_barrier()

        for b in range(2):
            colb, dstb, _, gb, gs, _ = bufs[b]
            pltpu.sync_copy(bsrc.at[c, s, p, pl.ds(b * CK, CK)], colb)
            pltpu.sync_copy(bdst.at[c, s, p, pl.ds(b * CK, CK)], dstb)
            pltpu.async_copy(uprev.at[colb], gb, gs)

        @pl.loop(0, nch // 2 - 1)
        def _chunks(h):
            for b in range(2):
                colb, dstb, dstl, gb, gs, ss = bufs[b]
                j = h * 2 + b
                _clamp_to_window(dstb, dstl, base, W)
                pltpu.make_async_copy(uprev.at[colb], gb, gs).wait()
                pltpu.async_copy(gb, acc.at[dstl], ss, add=True).wait()
                jn = (j + 2) * CK
                pltpu.sync_copy(bsrc.at[c, s, p, pl.ds(jn, CK)], colb)
                pltpu.sync_copy(bdst.at[c, s, p, pl.ds(jn, CK)], dstb)
                pltpu.async_copy(uprev.at[colb], gb, gs)

        for b in range(2):
            colb, dstb, dstl, gb, gs, ss = bufs[b]
            _clamp_to_window(dstb, dstl, base, W)
            pltpu.make_async_copy(uprev.at[colb], gb, gs).wait()
            pltpu.async_copy(gb, acc.at[dstl], ss, add=True).wait()

        plsc.subcore_barrier()

        g0 = c * SIDE_P + base + s * RT
        pltpu.sync_copy(dinv.at[pl.ds(g0, RT)], dinvb)

        @pl.loop(0, RT // 64)
        def _wb(t):
            pltpu.sync_copy(acc.at[pl.ds(s * RT + t * 64, 64)], wb)

            @pl.loop(0, 4)
            def _rows(g):
                dvv = dinvb[pl.ds(t * 64 + g * 16, 16)]
                for k in range(16):
                    r = g * 16 + k
                    dv = dvv[k]
                    for cc in range(D // 16):
                        sl = pl.ds(cc * 16, 16)
                        wb[r, sl] = wb[r, sl] * dv

            pltpu.sync_copy(wb, unext.at[pl.ds(g0 + t * 64, 64)])


def _layer_call(uprev, bsrc, bdst, cnts, dinv):
    kern = pl.kernel(
        _layer_body,
        out_type=jax.ShapeDtypeStruct((NP, D), _f32),
        mesh=_mesh(),
        compiler_params=pltpu.CompilerParams(use_tc_tiling_on_sc=False),
        scratch_types=[
            pltpu.VMEM_SHARED((W + 8, D), _f32),
            pltpu.VMEM((CK,), _i32),
            pltpu.VMEM((CK,), _i32),
            pltpu.VMEM((CK,), _i32),
            pltpu.VMEM((CK,), _i32),
            pltpu.VMEM((CK,), _i32),
            pltpu.VMEM((CK,), _i32),
            pltpu.VMEM((CK, D), _f32),
            pltpu.VMEM((CK, D), _f32),
            pltpu.VMEM((64, D), _f32),
            pltpu.VMEM((RT,), _f32),
            pltpu.VMEM((16,), _i32),
            pltpu.SemaphoreType.DMA,
            pltpu.SemaphoreType.DMA,
            pltpu.SemaphoreType.DMA,
            pltpu.SemaphoreType.DMA,
        ],
    )
    return kern(uprev, bsrc, bdst, cnts, dinv)


CAP_CH = NCH + 2
CAP = CAP_CH * CK


def _bucket_body(src, dst, bsrc, bdst, cnts,
                 colbb, dstbb, stages, staged, cntv, fill, gcnt):
    c = lax.axis_index("c")
    s = lax.axis_index("s")
    iot = lax.iota(_i32, 16)

    for p in range(PW):
        fill[p] = 0
        gcnt[p] = 0

    def _flush(p, pad_to_full):
        if pad_to_full:
            fv = jnp.broadcast_to(fill[p], (16,))
            for v in range(CK // 16):
                sl = pl.ds(v * 16, 16)
                keep = (iot + v * 16) < fv
                stages[p][sl] = jnp.where(keep, stages[p][sl], 0)
                staged[p][sl] = jnp.where(keep, staged[p][sl], -1)
        off = gcnt[p] * CK
        pltpu.sync_copy(stages[p].at[pl.ds(0, CK)], bsrc.at[c, s, p, pl.ds(off, CK)])
        pltpu.sync_copy(staged[p].at[pl.ds(0, CK)], bdst.at[c, s, p, pl.ds(off, CK)])
        gcnt[p] = gcnt[p] + 1

    def _shift_down(p):
        for v in range(CK // 16):
            sl = pl.ds(v * 16, 16)
            sh = pl.ds(CK + v * 16, 16)
            stages[p][sl] = stages[p][sh]
            staged[p][sl] = staged[p][sh]

    @pl.loop(0, NBLK)
    def _blocks(i):
        pltpu.sync_copy(src.at[c, s, pl.ds(i * BLK, BLK)], colbb)
        pltpu.sync_copy(dst.at[c, s, pl.ds(i * BLK, BLK)], dstbb)

        @pl.loop(0, BLK)
        def _chunk(k):
            for p in range(PW):
                base = p * W
                for v in range(CK // 16):
                    sl = pl.ds(v * 16, 16)
                    d = dstbb[k, sl]
                    loc = d - base
                    ok = (loc >= 0) & (loc < W)
                    mi = jnp.where(ok, 1, 0).astype(_i32)
                    csum = plsc.cumsum(mi)
                    pos = jnp.broadcast_to(fill[p], (16,)) + csum - 1
                    plsc.store_scatter(stages[p], [pos], colbb[k, sl], mask=ok)
                    plsc.store_scatter(staged[p], [pos], d, mask=ok)
                    fill[p] = fill[p] + csum[15]

                @pl.when(fill[p] >= CK)
                def _():
                    _flush(p, pad_to_full=False)
                    _shift_down(p)
                    fill[p] = fill[p] - CK

    for p in range(PW):
        _flush(p, pad_to_full=True)
        fill[p] = 0

        @pl.when(lax.rem(gcnt[p], 2) == 1)
        def _():
            _flush(p, pad_to_full=True)

    out = jnp.zeros((16,), _i32)
    for p in range(PW):
        out = jnp.where(iot == p, jnp.broadcast_to(gcnt[p], (16,)), out)
    cntv[pl.ds(0, 16)] = out
    pltpu.sync_copy(cntv, cnts.at[c, s])


def _bucket_call(src2, dst2):
    kern = pl.kernel(
        _bucket_body,
        out_type=(
            jax.ShapeDtypeStruct((NC, NS, PW, CAP), _i32),
            jax.ShapeDtypeStruct((NC, NS, PW, CAP), _i32),
            jax.ShapeDtypeStruct((NC, NS, 16), _i32),
        ),
        mesh=_mesh(),
        compiler_params=pltpu.CompilerParams(use_tc_tiling_on_sc=False,
                                             needs_layout_passes=False),
        scratch_types=[
            pltpu.VMEM((BLK, CK), _i32),
            pltpu.VMEM((BLK, CK), _i32),
            [pltpu.VMEM((2 * CK,), _i32) for _ in range(PW)],
            [pltpu.VMEM((2 * CK,), _i32) for _ in range(PW)],
            pltpu.VMEM((16,), _i32),
            pltpu.SMEM((PW,), _i32),
            pltpu.SMEM((PW,), _i32),
        ],
    )
    return kern(src2, dst2)


def _final_body(u0, u1, u2, u3, dsq16, uq, iq, out,
                uqb, iqb, gu0, gu1, gu2, gu3, gi0, gi1, gi2, gi3,
                du16, di16, ob16):
    c = lax.axis_index("c")
    s = lax.axis_index("s")
    wid = s * NC + c
    bq = 4096 // (NC * NS)
    sb = 64

    @pl.loop(0, bq // sb)
    def _sub(sr):
        base = wid * bq + sr * sb
        pltpu.sync_copy(uq.at[pl.ds(base, sb)], uqb)
        pltpu.sync_copy(iq.at[pl.ds(base, sb)], iqb)
        for tab, gb in ((u0, gu0), (u1, gu1), (u2, gu2), (u3, gu3)):
            pltpu.sync_copy(tab.at[uqb], gb)
        for tab, gb in ((u0, gi0), (u1, gi1), (u2, gi2), (u3, gi3)):
            pltpu.sync_copy(tab.at[iqb], gb)
        pltpu.sync_copy(dsq16.at[uqb], du16)
        pltpu.sync_copy(dsq16.at[iqb], di16)

        @pl.loop(0, sb)
        def _rows(r):
            tot = jnp.zeros((16,), _f32)
            for cc in range(D // 16):
                sl = pl.ds(cc * 16, 16)
                su = gu0[r, sl] + gu1[r, sl] + gu2[r, sl] + gu3[r, sl]
                si = gi0[r, sl] + gi1[r, sl] + gi2[r, sl] + gi3[r, sl]
                tot = tot + su * si
            ob16[r, pl.ds(0, 16)] = (tot * du16[r, pl.ds(0, 16)]
                                     * di16[r, pl.ds(0, 16)] * 0.0625)

        pltpu.sync_copy(ob16, out.at[pl.ds(base, sb)])


def _final_call(u0, u1, u2, u3, dsq16, uq, iq):
    bq = 4096 // (NC * NS)
    kern = pl.kernel(
        _final_body,
        out_type=jax.ShapeDtypeStruct((4096, 16), _f32),
        mesh=_mesh(),
        compiler_params=pltpu.CompilerParams(use_tc_tiling_on_sc=False),
        scratch_types=[
            pltpu.VMEM((64,), _i32),
            pltpu.VMEM((64,), _i32),
            pltpu.VMEM((64, D), _f32),
            pltpu.VMEM((64, D), _f32),
            pltpu.VMEM((64, D), _f32),
            pltpu.VMEM((64, D), _f32),
            pltpu.VMEM((64, D), _f32),
            pltpu.VMEM((64, D), _f32),
            pltpu.VMEM((64, D), _f32),
            pltpu.VMEM((64, D), _f32),
            pltpu.VMEM((64, 16), _f32),
            pltpu.VMEM((64, 16), _f32),
            pltpu.VMEM((64, 16), _f32),
        ],
    )
    return kern(u0, u1, u2, u3, dsq16, uq, iq)


def kernel(users, items, user_emb, item_emb, edge_u, edge_i, graph_vals):
    eu = edge_u.astype(_i32)
    ei = edge_i.astype(_i32)
    us = users.astype(_i32)
    it = items.astype(_i32)
    pad = EP_SIDE - E
    zpad = jnp.zeros((pad,), _i32)
    npad = jnp.full((pad,), -1, _i32)

    src_u = jnp.concatenate([ei + SIDE_P, zpad])
    dst_u = jnp.concatenate([eu, npad])
    src_i = jnp.concatenate([eu, zpad])
    dst_i = jnp.concatenate([ei, npad])
    src2 = jnp.stack([src_u, src_i]).reshape(NC, NS, NCH, CK)
    dst2 = jnp.stack([dst_u, dst_i]).reshape(NC, NS, NCH, CK)

    rpad = jnp.zeros((SIDE_P - SIDE, D), _f32)
    all_emb_p = jnp.concatenate([user_emb, rpad, item_emb, rpad], axis=0)

    deg = _deg_call(dst2)[:, 0]
    dm = jnp.maximum(deg, 1.0)
    dinv = 1.0 / dm
    dsqi2d = lax.rsqrt(dm)[:, None]
    dsq16 = jnp.broadcast_to(jnp.sqrt(dm)[:, None], (NP, 16))

    bsrc, bdst, cnts = _bucket_call(src2, dst2)

    u0 = _u0_call(all_emb_p, dsqi2d)
    u1 = _layer_call(u0, bsrc, bdst, cnts, dinv)
    u2 = _layer_call(u1, bsrc, bdst, cnts, dinv)
    u3 = _layer_call(u2, bsrc, bdst, cnts, dinv)

    return jnp.sum(_final_call(u0, u1, u2, u3, dsq16, us, it + SIDE_P), axis=1)
